# baseline (speedup 1.0000x reference)
import functools

import jax
import jax.numpy as jnp
from jax import lax
from jax.experimental import pallas as pl
from jax.experimental.pallas import tpu as pltpu
from jax.experimental.pallas import tpu_sc as plsc

N_NODES = 10000
D_FEAT = 256
DH = 128
N_SUBCORES = 16
N_CORES = 2
CHUNK = 128
K = 80
G = 8
NG = K // G
E_PAD = N_SUBCORES * K * CHUNK
ROWS_PER_TILE = 640
N_PAD = N_SUBCORES * ROWS_PER_TILE
CNT_W = 8
K_SPLIT = 40


def _sc_segment_sum_body(hcat, src4, dst3, z128, onesz8,
                         sums_out, cnts_out,
                         isrc0, isrc1, idst0, idst1, rows0, rows1,
                         ones_v, zbuf8,
                         sem_g0, sem_g1, sem_s0, sem_s1, sem_c,
                         acc, cacc):
    c = lax.axis_index("core")
    s = lax.axis_index("subcore")
    base = s * ROWS_PER_TILE
    nchunks = ROWS_PER_TILE // CHUNK
    isrc = (isrc0, isrc1)
    idst = (idst0, idst1)
    rows = (rows0, rows1)
    sem_g = (sem_g0, sem_g1)
    sem_s = (sem_s0, sem_s1)

    pltpu.sync_copy(onesz8.at[0], ones_v)
    pltpu.sync_copy(onesz8.at[1], zbuf8)
    pltpu.sync_copy(z128, rows0)

    @pl.loop(0, nchunks)
    def _(i):
        pltpu.sync_copy(rows0, acc.at[pl.ds(base + i * CHUNK, CHUNK)])
        pltpu.sync_copy(zbuf8, cacc.at[pl.ds(base + i * CHUNK, CHUNK)])
    plsc.subcore_barrier()

    @pl.loop(0, NG, step=2)
    def _(g2):
        for q in range(2):
            gidx = g2 + q
            counting = jnp.where(c == 0, gidx * G < K_SPLIT,
                                 gidx * G >= K_SPLIT)
            pltpu.sync_copy(src4.at[c, s, pl.ds(gidx * G, G)], isrc[q])
            pltpu.sync_copy(dst3.at[s, pl.ds(gidx * G, G)], idst[q])

            @pl.when(counting)
            def _():
                @pl.loop(0, G)
                def _(j):
                    pltpu.async_copy(ones_v, cacc.at[idst[q].at[j]], sem_c,
                                     add=True)

            @pl.loop(0, G, step=2)
            def _(j0):
                for p in range(2):
                    j = j0 + p
                    @pl.when(((g2 + q) * G + j0) > 0)
                    def _():
                        pltpu.make_async_copy(
                            rows[p], acc.at[pl.ds(base, CHUNK)],
                            sem_s[p]).wait()
                    pltpu.async_copy(hcat.at[isrc[q].at[j]], rows[p],
                                     sem_g[p])
                for p in range(2):
                    j = j0 + p
                    pltpu.make_async_copy(hcat.at[pl.ds(0, CHUNK)], rows[p],
                                          sem_g[p]).wait()
                    pltpu.async_copy(rows[p], acc.at[idst[q].at[j]],
                                     sem_s[p], add=True)

            @pl.when(counting)
            def _():
                @pl.loop(0, G)
                def _(j):
                    pltpu.make_async_copy(ones_v,
                                          cacc.at[pl.ds(base, CHUNK)],
                                          sem_c).wait()

    for p in range(2):
        pltpu.make_async_copy(rows[p], acc.at[pl.ds(base, CHUNK)],
                              sem_s[p]).wait()

    plsc.subcore_barrier()

    @pl.loop(0, nchunks)
    def _(i):
        pltpu.sync_copy(acc.at[pl.ds(base + i * CHUNK, CHUNK)], rows0)
        pltpu.sync_copy(rows0, sums_out.at[c, pl.ds(base + i * CHUNK, CHUNK)])
        pltpu.sync_copy(cacc.at[pl.ds(base + i * CHUNK, CHUNK)], zbuf8)
        pltpu.sync_copy(zbuf8, cnts_out.at[c, pl.ds(base + i * CHUNK, CHUNK)])


def _sc_segment_sum(hcat, src4, dst3, interpret=False):
    z128 = jnp.zeros((CHUNK, DH), jnp.float32)
    onesz8 = jnp.stack([jnp.ones((CHUNK, CNT_W), jnp.float32),
                        jnp.zeros((CHUNK, CNT_W), jnp.float32)])
    fn = pl.kernel(
        _sc_segment_sum_body,
        out_type=(
            jax.ShapeDtypeStruct((N_CORES, N_PAD, DH), jnp.float32),
            jax.ShapeDtypeStruct((N_CORES, N_PAD, CNT_W), jnp.float32),
        ),
        mesh=plsc.VectorSubcoreMesh(core_axis_name="core",
                                    subcore_axis_name="subcore",
                                    num_cores=N_CORES,
                                    num_subcores=N_SUBCORES),
        scratch_types=[
            pltpu.VMEM((G, CHUNK), jnp.int32),
            pltpu.VMEM((G, CHUNK), jnp.int32),
            pltpu.VMEM((G, CHUNK), jnp.int32),
            pltpu.VMEM((G, CHUNK), jnp.int32),
            pltpu.VMEM((CHUNK, DH), jnp.float32),
            pltpu.VMEM((CHUNK, DH), jnp.float32),
            pltpu.VMEM((CHUNK, CNT_W), jnp.float32),
            pltpu.VMEM((CHUNK, CNT_W), jnp.float32),
            pltpu.SemaphoreType.DMA,
            pltpu.SemaphoreType.DMA,
            pltpu.SemaphoreType.DMA,
            pltpu.SemaphoreType.DMA,
            pltpu.SemaphoreType.DMA,
            pltpu.VMEM_SHARED((N_PAD, DH), jnp.float32),
            pltpu.VMEM_SHARED((N_PAD, CNT_W), jnp.float32),
        ],
        compiler_params=pltpu.CompilerParams(use_tc_tiling_on_sc=False),
        interpret=interpret,
    )
    return fn(hcat, src4, dst3, z128, onesz8)


M_BLK = 400


def _tc_linear_body(h_ref, slo_ref, shi_ref, c0_ref, c1_ref,
                    w1_ref, w2a_ref, w2b_ref, b_ref, o_ref):
    cnt = c0_ref[0][:, 0:1] + c1_ref[0][:, 0:1]
    recip = 1.0 / jnp.maximum(cnt, 1.0)
    self_part = jnp.dot(h_ref[...], w1_ref[...],
                        preferred_element_type=jnp.float32)
    agg = jnp.dot(slo_ref[0], w2a_ref[...],
                  preferred_element_type=jnp.float32)
    agg = agg + jnp.dot(shi_ref[0], w2b_ref[...],
                        preferred_element_type=jnp.float32)
    o_ref[...] = self_part + agg * recip + b_ref[...]


def _tc_linear(h, sums, cnts, w1t, w2at, w2bt, b2, interpret=False):
    grid = (N_NODES // M_BLK,)
    return pl.pallas_call(
        _tc_linear_body,
        grid=grid,
        in_specs=[
            pl.BlockSpec((M_BLK, D_FEAT), lambda i: (i, 0)),
            pl.BlockSpec((1, M_BLK, DH), lambda i: (0, i, 0)),
            pl.BlockSpec((1, M_BLK, DH), lambda i: (1, i, 0)),
            pl.BlockSpec((1, M_BLK, CNT_W), lambda i: (0, i, 0)),
            pl.BlockSpec((1, M_BLK, CNT_W), lambda i: (1, i, 0)),
            pl.BlockSpec((D_FEAT, D_FEAT), lambda i: (0, 0)),
            pl.BlockSpec((DH, D_FEAT), lambda i: (0, 0)),
            pl.BlockSpec((DH, D_FEAT), lambda i: (0, 0)),
            pl.BlockSpec((1, D_FEAT), lambda i: (0, 0)),
        ],
        out_specs=pl.BlockSpec((M_BLK, D_FEAT), lambda i: (i, 0)),
        out_shape=jax.ShapeDtypeStruct((N_NODES, D_FEAT), jnp.float32),
        interpret=interpret,
    )(h, sums, sums, cnts, cnts, w1t, w2at, w2bt, b2)


def kernel(h, edge_index, W, b, interpret=False):
    src = edge_index[0].astype(jnp.int32)
    dst = edge_index[1].astype(jnp.int32)
    e = src.shape[0]
    pad = E_PAD - e
    src_p = jnp.concatenate([src, jnp.zeros((pad,), jnp.int32)])
    dst_p = jnp.concatenate([dst, jnp.full((pad,), N_NODES, jnp.int32)])
    src3 = src_p.reshape(N_SUBCORES, K, CHUNK)
    src4 = jnp.stack([src3, src3 + N_NODES])
    dst3 = dst_p.reshape(N_SUBCORES, K, CHUNK)
    hcat = jnp.concatenate([h[:, :DH], h[:, DH:]], axis=0)

    sums, cnts = _sc_segment_sum(hcat, src4, dst3, interpret=interpret)

    wt = W.T
    w1t = wt[:D_FEAT]
    w2at = wt[D_FEAT:D_FEAT + DH]
    w2bt = wt[D_FEAT + DH:]
    b2 = b.reshape(1, D_FEAT)
    return _tc_linear(h, sums, cnts, w1t, w2at, w2bt, b2, interpret=interpret)


# speedup vs baseline: 1.3063x; 1.3063x over previous
import jax
import jax.numpy as jnp
from jax import lax
from jax.experimental import pallas as pl
from jax.experimental.pallas import tpu as pltpu
from jax.experimental.pallas import tpu_sc as plsc

N_NODES = 10000
D_FEAT = 256
DQ = 64
N_SUBCORES = 16
N_CORES = 2
CHUNK = 128
K = 80
G = 8
NG = K // G
E_PAD = N_SUBCORES * K * CHUNK
ROWS_PER_TILE = 640
N_PAD = N_SUBCORES * ROWS_PER_TILE
CNT_W = 8
K_SPLIT = 40
NCH = ROWS_PER_TILE // CHUNK


def _sc_segment_sum_body(hq4, src3, dst3, onesz8,
                         sums_out, cnts_out,
                         isrc0, isrc1, idst0, idst1, rows0, rows1,
                         ones_v, zbuf8,
                         sem_g0, sem_g1, sem_s0, sem_s1, sem_c,
                         htab, acc, cacc):
    c = lax.axis_index("core")
    s = lax.axis_index("subcore")
    base = s * ROWS_PER_TILE
    isrc = (isrc0, isrc1)
    idst = (idst0, idst1)
    rows = (rows0, rows1)
    sem_g = (sem_g0, sem_g1)
    sem_s = (sem_s0, sem_s1)

    pltpu.sync_copy(onesz8.at[0], ones_v)
    pltpu.sync_copy(onesz8.at[1], zbuf8)

    for t in range(2):
        sl = 2 * c + t

        @pl.loop(0, NCH)
        def _(i):
            off = base + i * CHUNK
            pltpu.sync_copy(hq4.at[sl, pl.ds(off, CHUNK)], rows0)
            pltpu.sync_copy(rows0, htab.at[pl.ds(off, CHUNK)])
            pltpu.sync_copy(hq4.at[4, pl.ds(off, CHUNK)], rows1)
            pltpu.sync_copy(rows1, acc.at[pl.ds(off, CHUNK)])
            if t == 0:
                pltpu.sync_copy(zbuf8, cacc.at[pl.ds(off, CHUNK)])
        plsc.subcore_barrier()

        @pl.loop(0, NG, step=2)
        def _(g2):
            for q in range(2):
                gidx = g2 + q
                counting = jnp.where(c == 0, gidx * G < K_SPLIT,
                                     gidx * G >= K_SPLIT)
                pltpu.sync_copy(src3.at[s, pl.ds(gidx * G, G)], isrc[q])
                pltpu.sync_copy(dst3.at[s, pl.ds(gidx * G, G)], idst[q])

                if t == 0:
                    @pl.when(counting)
                    def _():
                        @pl.loop(0, G)
                        def _(j):
                            pltpu.async_copy(ones_v, cacc.at[idst[q].at[j]],
                                             sem_c, add=True)

                @pl.loop(0, G, step=2)
                def _(j0):
                    for p in range(2):
                        j = j0 + p
                        @pl.when(((g2 + q) * G + j0) > 0)
                        def _():
                            pltpu.make_async_copy(
                                rows[p], acc.at[pl.ds(base, CHUNK)],
                                sem_s[p]).wait()
                        pltpu.async_copy(htab.at[isrc[q].at[j]], rows[p],
                                         sem_g[p])
                        pltpu.make_async_copy(htab.at[pl.ds(0, CHUNK)],
                                              rows[p], sem_g[p]).wait()
                        pltpu.async_copy(rows[p], acc.at[idst[q].at[j]],
                                         sem_s[p], add=True)

                if t == 0:
                    @pl.when(counting)
                    def _():
                        @pl.loop(0, G)
                        def _(j):
                            pltpu.make_async_copy(
                                ones_v, cacc.at[pl.ds(base, CHUNK)],
                                sem_c).wait()

        for p in range(2):
            pltpu.make_async_copy(rows[p], acc.at[pl.ds(base, CHUNK)],
                                  sem_s[p]).wait()
        plsc.subcore_barrier()

        @pl.loop(0, NCH)
        def _(i):
            off = base + i * CHUNK
            pltpu.sync_copy(acc.at[pl.ds(off, CHUNK)], rows0)
            pltpu.sync_copy(rows0, sums_out.at[sl, pl.ds(off, CHUNK)])
            if t == 1:
                pltpu.sync_copy(cacc.at[pl.ds(off, CHUNK)], zbuf8)
                pltpu.sync_copy(zbuf8, cnts_out.at[c, pl.ds(off, CHUNK)])


def _sc_segment_sum(hq4, src3, dst3, interpret=False):
    onesz8 = jnp.stack([jnp.ones((CHUNK, CNT_W), jnp.float32),
                        jnp.zeros((CHUNK, CNT_W), jnp.float32)])
    fn = pl.kernel(
        _sc_segment_sum_body,
        out_type=(
            jax.ShapeDtypeStruct((4, N_PAD, DQ), jnp.float32),
            jax.ShapeDtypeStruct((N_CORES, N_PAD, CNT_W), jnp.float32),
        ),
        mesh=plsc.VectorSubcoreMesh(core_axis_name="core",
                                    subcore_axis_name="subcore",
                                    num_cores=N_CORES,
                                    num_subcores=N_SUBCORES),
        scratch_types=[
            pltpu.VMEM((G, CHUNK), jnp.int32),
            pltpu.VMEM((G, CHUNK), jnp.int32),
            pltpu.VMEM((G, CHUNK), jnp.int32),
            pltpu.VMEM((G, CHUNK), jnp.int32),
            pltpu.VMEM((CHUNK, DQ), jnp.float32),
            pltpu.VMEM((CHUNK, DQ), jnp.float32),
            pltpu.VMEM((CHUNK, CNT_W), jnp.float32),
            pltpu.VMEM((CHUNK, CNT_W), jnp.float32),
            pltpu.SemaphoreType.DMA,
            pltpu.SemaphoreType.DMA,
            pltpu.SemaphoreType.DMA,
            pltpu.SemaphoreType.DMA,
            pltpu.SemaphoreType.DMA,
            pltpu.VMEM_SHARED((N_PAD, DQ), jnp.float32),
            pltpu.VMEM_SHARED((N_PAD, DQ), jnp.float32),
            pltpu.VMEM_SHARED((N_PAD, CNT_W), jnp.float32),
        ],
        compiler_params=pltpu.CompilerParams(use_tc_tiling_on_sc=False),
        interpret=interpret,
    )
    return fn(hq4, src3, dst3, onesz8)


M_BLK = 400


def _tc_linear_body(h_ref, s0_ref, s1_ref, s2_ref, s3_ref, c0_ref, c1_ref,
                    w1_ref, w2_ref, b_ref, o_ref):
    cnt = c0_ref[0][:, 0:1] + c1_ref[0][:, 0:1]
    recip = 1.0 / jnp.maximum(cnt, 1.0)
    out = jnp.dot(h_ref[...], w1_ref[...], preferred_element_type=jnp.float32)
    agg = jnp.dot(s0_ref[0], w2_ref[pl.ds(0, DQ)],
                  preferred_element_type=jnp.float32)
    agg += jnp.dot(s1_ref[0], w2_ref[pl.ds(DQ, DQ)],
                   preferred_element_type=jnp.float32)
    agg += jnp.dot(s2_ref[0], w2_ref[pl.ds(2 * DQ, DQ)],
                   preferred_element_type=jnp.float32)
    agg += jnp.dot(s3_ref[0], w2_ref[pl.ds(3 * DQ, DQ)],
                   preferred_element_type=jnp.float32)
    o_ref[...] = out + agg * recip + b_ref[...]


def _tc_linear(h, sums, cnts, w1t, w2t, b2, interpret=False):
    grid = (N_NODES // M_BLK,)
    return pl.pallas_call(
        _tc_linear_body,
        grid=grid,
        in_specs=[
            pl.BlockSpec((M_BLK, D_FEAT), lambda i: (i, 0)),
            pl.BlockSpec((1, M_BLK, DQ), lambda i: (0, i, 0)),
            pl.BlockSpec((1, M_BLK, DQ), lambda i: (1, i, 0)),
            pl.BlockSpec((1, M_BLK, DQ), lambda i: (2, i, 0)),
            pl.BlockSpec((1, M_BLK, DQ), lambda i: (3, i, 0)),
            pl.BlockSpec((1, M_BLK, CNT_W), lambda i: (0, i, 0)),
            pl.BlockSpec((1, M_BLK, CNT_W), lambda i: (1, i, 0)),
            pl.BlockSpec((D_FEAT, D_FEAT), lambda i: (0, 0)),
            pl.BlockSpec((D_FEAT, D_FEAT), lambda i: (0, 0)),
            pl.BlockSpec((1, D_FEAT), lambda i: (0, 0)),
        ],
        out_specs=pl.BlockSpec((M_BLK, D_FEAT), lambda i: (i, 0)),
        out_shape=jax.ShapeDtypeStruct((N_NODES, D_FEAT), jnp.float32),
        interpret=interpret,
    )(h, sums, sums, sums, sums, cnts, cnts, w1t, w2t, b2)


def kernel(h, edge_index, W, b, interpret=False):
    src = edge_index[0].astype(jnp.int32)
    dst = edge_index[1].astype(jnp.int32)
    e = src.shape[0]
    pad = E_PAD - e
    src_p = jnp.concatenate([src, jnp.zeros((pad,), jnp.int32)])
    dst_p = jnp.concatenate([dst, jnp.full((pad,), N_NODES, jnp.int32)])
    src3 = src_p.reshape(N_SUBCORES, K, CHUNK)
    dst3 = dst_p.reshape(N_SUBCORES, K, CHUNK)
    hq = jnp.transpose(h.reshape(N_NODES, 4, DQ), (1, 0, 2))
    hq4 = jnp.concatenate(
        [hq, jnp.zeros((1, N_NODES, DQ), jnp.float32)], axis=0)
    hq4 = jnp.concatenate(
        [hq4, jnp.zeros((5, N_PAD - N_NODES, DQ), jnp.float32)], axis=1)

    sums, cnts = _sc_segment_sum(hq4, src3, dst3, interpret=interpret)

    wt = W.T
    w1t = wt[:D_FEAT]
    w2t = wt[D_FEAT:]
    b2 = b.reshape(1, D_FEAT)
    return _tc_linear(h, sums, cnts, w1t, w2t, b2, interpret=interpret)


# speedup vs baseline: 1.5600x; 1.1942x over previous
import jax
import jax.numpy as jnp
from jax import lax
from jax.experimental import pallas as pl
from jax.experimental.pallas import tpu as pltpu
from jax.experimental.pallas import tpu_sc as plsc

N_NODES = 10000
D_FEAT = 256
DQ = 64
N_SUBCORES = 16
N_CORES = 2
CHUNK = 128
K = 80
G = 8
NG = K // G
E_PAD = N_SUBCORES * K * CHUNK
ROWS_PER_TILE = 640
N_PAD = N_SUBCORES * ROWS_PER_TILE
CNT_W = 8
K_SPLIT = 40
NCH = ROWS_PER_TILE // CHUNK


def _sc_segment_sum_body(h, src3, dst3, onesz8, z64,
                         sums_out, cnts_out,
                         isrc0, isrc1, idst0, idst1, rows0, rows1,
                         ones_v, zbuf8,
                         sem_g0, sem_g1, sem_s0, sem_s1, sem_c,
                         htab, acc, cacc):
    c = lax.axis_index("core")
    s = lax.axis_index("subcore")
    base = s * ROWS_PER_TILE
    isrc = (isrc0, isrc1)
    idst = (idst0, idst1)
    rows = (rows0, rows1)
    sem_g = (sem_g0, sem_g1)
    sem_s = (sem_s0, sem_s1)

    pltpu.sync_copy(onesz8.at[0], ones_v)
    pltpu.sync_copy(onesz8.at[1], zbuf8)

    for t in range(2):
        sl = 2 * c + t

        pltpu.sync_copy(z64, rows1)
        tail = N_NODES % CHUNK

        @pl.loop(0, NCH)
        def _(i):
            off = base + i * CHUNK
            pltpu.sync_copy(rows1, acc.at[pl.ds(off, CHUNK)])
            if t == 0:
                pltpu.sync_copy(zbuf8, cacc.at[pl.ds(off, CHUNK)])

            @pl.when(off + CHUNK <= N_NODES)
            def _():
                pltpu.sync_copy(h.at[pl.ds(off, CHUNK), pl.ds(sl * DQ, DQ)],
                                rows0)
                pltpu.sync_copy(rows0, htab.at[pl.ds(off, CHUNK)])

            @pl.when((off < N_NODES) & (off + CHUNK > N_NODES))
            def _():
                pltpu.sync_copy(h.at[pl.ds(off, tail), pl.ds(sl * DQ, DQ)],
                                rows0.at[pl.ds(0, tail)])
                pltpu.sync_copy(rows0.at[pl.ds(0, tail)],
                                htab.at[pl.ds(off, tail)])
        plsc.subcore_barrier()

        @pl.loop(0, NG, step=2)
        def _(g2):
            for q in range(2):
                gidx = g2 + q
                counting = jnp.where(c == 0, gidx * G < K_SPLIT,
                                     gidx * G >= K_SPLIT)
                pltpu.sync_copy(src3.at[s, pl.ds(gidx * G, G)], isrc[q])
                pltpu.sync_copy(dst3.at[s, pl.ds(gidx * G, G)], idst[q])

                if t == 0:
                    @pl.when(counting)
                    def _():
                        @pl.loop(0, G)
                        def _(j):
                            pltpu.async_copy(ones_v, cacc.at[idst[q].at[j]],
                                             sem_c, add=True)

                @pl.loop(0, G, step=2)
                def _(j0):
                    for p in range(2):
                        j = j0 + p
                        @pl.when(((g2 + q) * G + j0) > 0)
                        def _():
                            pltpu.make_async_copy(
                                rows[p], acc.at[pl.ds(base, CHUNK)],
                                sem_s[p]).wait()
                        pltpu.async_copy(htab.at[isrc[q].at[j]], rows[p],
                                         sem_g[p])
                        pltpu.make_async_copy(htab.at[pl.ds(0, CHUNK)],
                                              rows[p], sem_g[p]).wait()
                        pltpu.async_copy(rows[p], acc.at[idst[q].at[j]],
                                         sem_s[p], add=True)

                if t == 0:
                    @pl.when(counting)
                    def _():
                        @pl.loop(0, G)
                        def _(j):
                            pltpu.make_async_copy(
                                ones_v, cacc.at[pl.ds(base, CHUNK)],
                                sem_c).wait()

        for p in range(2):
            pltpu.make_async_copy(rows[p], acc.at[pl.ds(base, CHUNK)],
                                  sem_s[p]).wait()
        plsc.subcore_barrier()

        @pl.loop(0, NCH)
        def _(i):
            off = base + i * CHUNK
            pltpu.sync_copy(acc.at[pl.ds(off, CHUNK)], rows0)
            pltpu.sync_copy(rows0, sums_out.at[sl, pl.ds(off, CHUNK)])
            if t == 1:
                pltpu.sync_copy(cacc.at[pl.ds(off, CHUNK)], zbuf8)
                pltpu.sync_copy(zbuf8, cnts_out.at[c, pl.ds(off, CHUNK)])


def _sc_segment_sum(h, src3, dst3, interpret=False):
    onesz8 = jnp.stack([jnp.ones((CHUNK, CNT_W), jnp.float32),
                        jnp.zeros((CHUNK, CNT_W), jnp.float32)])
    z64 = jnp.zeros((CHUNK, DQ), jnp.float32)
    fn = pl.kernel(
        _sc_segment_sum_body,
        out_type=(
            jax.ShapeDtypeStruct((4, N_PAD, DQ), jnp.float32),
            jax.ShapeDtypeStruct((N_CORES, N_PAD, CNT_W), jnp.float32),
        ),
        mesh=plsc.VectorSubcoreMesh(core_axis_name="core",
                                    subcore_axis_name="subcore",
                                    num_cores=N_CORES,
                                    num_subcores=N_SUBCORES),
        scratch_types=[
            pltpu.VMEM((G, CHUNK), jnp.int32),
            pltpu.VMEM((G, CHUNK), jnp.int32),
            pltpu.VMEM((G, CHUNK), jnp.int32),
            pltpu.VMEM((G, CHUNK), jnp.int32),
            pltpu.VMEM((CHUNK, DQ), jnp.float32),
            pltpu.VMEM((CHUNK, DQ), jnp.float32),
            pltpu.VMEM((CHUNK, CNT_W), jnp.float32),
            pltpu.VMEM((CHUNK, CNT_W), jnp.float32),
            pltpu.SemaphoreType.DMA,
            pltpu.SemaphoreType.DMA,
            pltpu.SemaphoreType.DMA,
            pltpu.SemaphoreType.DMA,
            pltpu.SemaphoreType.DMA,
            pltpu.VMEM_SHARED((N_PAD, DQ), jnp.float32),
            pltpu.VMEM_SHARED((N_PAD, DQ), jnp.float32),
            pltpu.VMEM_SHARED((N_PAD, CNT_W), jnp.float32),
        ],
        compiler_params=pltpu.CompilerParams(use_tc_tiling_on_sc=False),
        interpret=interpret,
    )
    return fn(h, src3, dst3, onesz8, z64)


M_BLK = 400


def _tc_self_body(h_ref, w1_ref, b_ref, o_ref):
    o_ref[...] = jnp.dot(h_ref[...], w1_ref[...],
                         preferred_element_type=jnp.float32) + b_ref[...]


def _tc_self(h, w1t, b2, interpret=False):
    return pl.pallas_call(
        _tc_self_body,
        grid=(N_NODES // M_BLK,),
        in_specs=[
            pl.BlockSpec((M_BLK, D_FEAT), lambda i: (i, 0)),
            pl.BlockSpec((D_FEAT, D_FEAT), lambda i: (0, 0)),
            pl.BlockSpec((1, D_FEAT), lambda i: (0, 0)),
        ],
        out_specs=pl.BlockSpec((M_BLK, D_FEAT), lambda i: (i, 0)),
        out_shape=jax.ShapeDtypeStruct((N_NODES, D_FEAT), jnp.float32),
        interpret=interpret,
    )(h, w1t, b2)


def _tc_combine_body(self_ref, s0_ref, s1_ref, s2_ref, s3_ref,
                     c0_ref, c1_ref, w2_ref, o_ref):
    cnt = c0_ref[0][:, 0:1] + c1_ref[0][:, 0:1]
    recip = 1.0 / jnp.maximum(cnt, 1.0)
    agg = jnp.dot(s0_ref[0], w2_ref[pl.ds(0, DQ)],
                  preferred_element_type=jnp.float32)
    agg += jnp.dot(s1_ref[0], w2_ref[pl.ds(DQ, DQ)],
                   preferred_element_type=jnp.float32)
    agg += jnp.dot(s2_ref[0], w2_ref[pl.ds(2 * DQ, DQ)],
                   preferred_element_type=jnp.float32)
    agg += jnp.dot(s3_ref[0], w2_ref[pl.ds(3 * DQ, DQ)],
                   preferred_element_type=jnp.float32)
    o_ref[...] = self_ref[...] + agg * recip


def _tc_combine(self_part, sums, cnts, w2t, interpret=False):
    return pl.pallas_call(
        _tc_combine_body,
        grid=(N_NODES // M_BLK,),
        in_specs=[
            pl.BlockSpec((M_BLK, D_FEAT), lambda i: (i, 0)),
            pl.BlockSpec((1, M_BLK, DQ), lambda i: (0, i, 0)),
            pl.BlockSpec((1, M_BLK, DQ), lambda i: (1, i, 0)),
            pl.BlockSpec((1, M_BLK, DQ), lambda i: (2, i, 0)),
            pl.BlockSpec((1, M_BLK, DQ), lambda i: (3, i, 0)),
            pl.BlockSpec((1, M_BLK, CNT_W), lambda i: (0, i, 0)),
            pl.BlockSpec((1, M_BLK, CNT_W), lambda i: (1, i, 0)),
            pl.BlockSpec((D_FEAT, D_FEAT), lambda i: (0, 0)),
        ],
        out_specs=pl.BlockSpec((M_BLK, D_FEAT), lambda i: (i, 0)),
        out_shape=jax.ShapeDtypeStruct((N_NODES, D_FEAT), jnp.float32),
        interpret=interpret,
    )(self_part, sums, sums, sums, sums, cnts, cnts, w2t)


def kernel(h, edge_index, W, b, interpret=False):
    src = edge_index[0].astype(jnp.int32)
    dst = edge_index[1].astype(jnp.int32)
    e = src.shape[0]
    pad = E_PAD - e
    src_p = jnp.concatenate([src, jnp.zeros((pad,), jnp.int32)])
    dst_p = jnp.concatenate([dst, jnp.full((pad,), N_NODES, jnp.int32)])
    src3 = src_p.reshape(N_SUBCORES, K, CHUNK)
    dst3 = dst_p.reshape(N_SUBCORES, K, CHUNK)

    sums, cnts = _sc_segment_sum(h, src3, dst3, interpret=interpret)

    wt = W.T
    w1t = wt[:D_FEAT]
    w2t = wt[D_FEAT:]
    b2 = b.reshape(1, D_FEAT)
    self_part = _tc_self(h, w1t, b2, interpret=interpret)
    return _tc_combine(self_part, sums, cnts, w2t, interpret=interpret)


# speedup vs baseline: 1.5899x; 1.0192x over previous
import jax
import jax.numpy as jnp
from jax import lax
from jax.experimental import pallas as pl
from jax.experimental.pallas import tpu as pltpu
from jax.experimental.pallas import tpu_sc as plsc

N_NODES = 10000
D_FEAT = 256
DQ = 64
N_SUBCORES = 16
N_CORES = 2
CHUNK = 128
K = 80
G = 8
NG = K // G
E_PAD = N_SUBCORES * K * CHUNK
ROWS_PER_TILE = 640
N_PAD = N_SUBCORES * ROWS_PER_TILE
CNT_W = 8
K_SPLIT = 40
NCH = ROWS_PER_TILE // CHUNK


def _sc_segment_sum_body(h, src3, dst3, onesz8, z64,
                         sums_out, cnts_out,
                         isrc0, isrc1, idst0, idst1, rows0, rows1, zrows,
                         ones_v, zbuf8,
                         sem_g0, sem_g1, sem_s0, sem_s1, sem_c,
                         htab, acc, cacc):
    c = lax.axis_index("core")
    s = lax.axis_index("subcore")
    base = s * ROWS_PER_TILE
    isrc = (isrc0, isrc1)
    idst = (idst0, idst1)
    rows = (rows0, rows1)
    sem_g = (sem_g0, sem_g1)
    sem_s = (sem_s0, sem_s1)

    pltpu.sync_copy(onesz8.at[0], ones_v)
    pltpu.sync_copy(onesz8.at[1], zbuf8)

    for t in range(2):
        sl = 2 * c + t

        pltpu.sync_copy(z64, zrows)
        tail = N_NODES % CHUNK

        @pl.loop(0, NCH)
        def _(i):
            off = base + i * CHUNK
            pltpu.async_copy(zrows, acc.at[pl.ds(off, CHUNK)], sem_c)
            if t == 0:
                pltpu.async_copy(zbuf8, cacc.at[pl.ds(off, CHUNK)], sem_c)

        @pl.loop(0, NCH, step=1)
        def _(i):
            off = base + i * CHUNK

            @pl.when(off + CHUNK <= N_NODES)
            def _():
                pltpu.async_copy(h.at[pl.ds(off, CHUNK), pl.ds(sl * DQ, DQ)],
                                 rows0, sem_g0)
                pltpu.make_async_copy(
                    h.at[pl.ds(0, CHUNK), pl.ds(0, DQ)], rows0,
                    sem_g0).wait()
                pltpu.sync_copy(rows0, htab.at[pl.ds(off, CHUNK)])

            @pl.when((off < N_NODES) & (off + CHUNK > N_NODES))
            def _():
                pltpu.sync_copy(h.at[pl.ds(off, tail), pl.ds(sl * DQ, DQ)],
                                rows0.at[pl.ds(0, tail)])
                pltpu.sync_copy(rows0.at[pl.ds(0, tail)],
                                htab.at[pl.ds(off, tail)])

        @pl.loop(0, NCH)
        def _(i):
            pltpu.make_async_copy(zrows, acc.at[pl.ds(base, CHUNK)],
                                  sem_c).wait()
            if t == 0:
                pltpu.make_async_copy(zbuf8, cacc.at[pl.ds(base, CHUNK)],
                                      sem_c).wait()
        plsc.subcore_barrier()

        @pl.loop(0, NG, step=2)
        def _(g2):
            for q in range(2):
                gidx = g2 + q
                counting = jnp.where(c == 0, gidx * G < K_SPLIT,
                                     gidx * G >= K_SPLIT)
                pltpu.sync_copy(src3.at[s, pl.ds(gidx * G, G)], isrc[q])
                pltpu.sync_copy(dst3.at[s, pl.ds(gidx * G, G)], idst[q])

                if t == 0:
                    @pl.when(counting)
                    def _():
                        @pl.loop(0, G)
                        def _(j):
                            pltpu.async_copy(ones_v, cacc.at[idst[q].at[j]],
                                             sem_c, add=True)

                @pl.loop(0, G, step=2)
                def _(j0):
                    for p in range(2):
                        j = j0 + p
                        @pl.when(((g2 + q) * G + j0) > 0)
                        def _():
                            pltpu.make_async_copy(
                                rows[p], acc.at[pl.ds(base, CHUNK)],
                                sem_s[p]).wait()
                        pltpu.async_copy(htab.at[isrc[q].at[j]], rows[p],
                                         sem_g[p])
                        pltpu.make_async_copy(htab.at[pl.ds(0, CHUNK)],
                                              rows[p], sem_g[p]).wait()
                        pltpu.async_copy(rows[p], acc.at[idst[q].at[j]],
                                         sem_s[p], add=True)

                if t == 0:
                    @pl.when(counting)
                    def _():
                        @pl.loop(0, G)
                        def _(j):
                            pltpu.make_async_copy(
                                ones_v, cacc.at[pl.ds(base, CHUNK)],
                                sem_c).wait()

        for p in range(2):
            pltpu.make_async_copy(rows[p], acc.at[pl.ds(base, CHUNK)],
                                  sem_s[p]).wait()
        plsc.subcore_barrier()

        @pl.loop(0, NCH - 1, step=2)
        def _(i0):
            for p in range(2):
                i = i0 + p

                @pl.when(i0 > 0)
                def _():
                    pltpu.make_async_copy(
                        rows[p], sums_out.at[sl, pl.ds(base, CHUNK)],
                        sem_s[p]).wait()
                off = base + i * CHUNK
                pltpu.sync_copy(acc.at[pl.ds(off, CHUNK)], rows[p])
                pltpu.async_copy(rows[p], sums_out.at[sl, pl.ds(off, CHUNK)],
                                 sem_s[p])
        offl = base + (NCH - 1) * CHUNK
        pltpu.make_async_copy(rows0, sums_out.at[sl, pl.ds(base, CHUNK)],
                              sem_s0).wait()
        pltpu.sync_copy(acc.at[pl.ds(offl, CHUNK)], rows0)
        pltpu.async_copy(rows0, sums_out.at[sl, pl.ds(offl, CHUNK)], sem_s0)
        if t == 1:
            @pl.loop(0, NCH)
            def _(i):
                off = base + i * CHUNK
                pltpu.sync_copy(cacc.at[pl.ds(off, CHUNK)], zbuf8)
                pltpu.sync_copy(zbuf8, cnts_out.at[c, pl.ds(off, CHUNK)])
        for p in range(2):
            pltpu.make_async_copy(rows[p],
                                  sums_out.at[sl, pl.ds(base, CHUNK)],
                                  sem_s[p]).wait()


def _sc_segment_sum(h, src3, dst3, interpret=False):
    onesz8 = jnp.stack([jnp.ones((CHUNK, CNT_W), jnp.float32),
                        jnp.zeros((CHUNK, CNT_W), jnp.float32)])
    z64 = jnp.zeros((CHUNK, DQ), jnp.float32)
    fn = pl.kernel(
        _sc_segment_sum_body,
        out_type=(
            jax.ShapeDtypeStruct((4, N_PAD, DQ), jnp.float32),
            jax.ShapeDtypeStruct((N_CORES, N_PAD, CNT_W), jnp.float32),
        ),
        mesh=plsc.VectorSubcoreMesh(core_axis_name="core",
                                    subcore_axis_name="subcore",
                                    num_cores=N_CORES,
                                    num_subcores=N_SUBCORES),
        scratch_types=[
            pltpu.VMEM((G, CHUNK), jnp.int32),
            pltpu.VMEM((G, CHUNK), jnp.int32),
            pltpu.VMEM((G, CHUNK), jnp.int32),
            pltpu.VMEM((G, CHUNK), jnp.int32),
            pltpu.VMEM((CHUNK, DQ), jnp.float32),
            pltpu.VMEM((CHUNK, DQ), jnp.float32),
            pltpu.VMEM((CHUNK, DQ), jnp.float32),
            pltpu.VMEM((CHUNK, CNT_W), jnp.float32),
            pltpu.VMEM((CHUNK, CNT_W), jnp.float32),
            pltpu.SemaphoreType.DMA,
            pltpu.SemaphoreType.DMA,
            pltpu.SemaphoreType.DMA,
            pltpu.SemaphoreType.DMA,
            pltpu.SemaphoreType.DMA,
            pltpu.VMEM_SHARED((N_PAD, DQ), jnp.float32),
            pltpu.VMEM_SHARED((N_PAD, DQ), jnp.float32),
            pltpu.VMEM_SHARED((N_PAD, CNT_W), jnp.float32),
        ],
        compiler_params=pltpu.CompilerParams(use_tc_tiling_on_sc=False),
        interpret=interpret,
    )
    return fn(h, src3, dst3, onesz8, z64)


M_BLK = 400


def _tc_self_body(h_ref, w1_ref, b_ref, o_ref):
    o_ref[...] = jnp.dot(h_ref[...], w1_ref[...],
                         preferred_element_type=jnp.float32) + b_ref[...]


def _tc_self(h, w1t, b2, interpret=False):
    return pl.pallas_call(
        _tc_self_body,
        grid=(N_NODES // M_BLK,),
        in_specs=[
            pl.BlockSpec((M_BLK, D_FEAT), lambda i: (i, 0)),
            pl.BlockSpec((D_FEAT, D_FEAT), lambda i: (0, 0)),
            pl.BlockSpec((1, D_FEAT), lambda i: (0, 0)),
        ],
        out_specs=pl.BlockSpec((M_BLK, D_FEAT), lambda i: (i, 0)),
        out_shape=jax.ShapeDtypeStruct((N_NODES, D_FEAT), jnp.float32),
        interpret=interpret,
    )(h, w1t, b2)


def _tc_combine_body(self_ref, s0_ref, s1_ref, s2_ref, s3_ref,
                     c0_ref, c1_ref, w2_ref, o_ref):
    cnt = c0_ref[0][:, 0:1] + c1_ref[0][:, 0:1]
    recip = 1.0 / jnp.maximum(cnt, 1.0)
    agg = jnp.dot(s0_ref[0], w2_ref[pl.ds(0, DQ)],
                  preferred_element_type=jnp.float32)
    agg += jnp.dot(s1_ref[0], w2_ref[pl.ds(DQ, DQ)],
                   preferred_element_type=jnp.float32)
    agg += jnp.dot(s2_ref[0], w2_ref[pl.ds(2 * DQ, DQ)],
                   preferred_element_type=jnp.float32)
    agg += jnp.dot(s3_ref[0], w2_ref[pl.ds(3 * DQ, DQ)],
                   preferred_element_type=jnp.float32)
    o_ref[...] = self_ref[...] + agg * recip


def _tc_combine(self_part, sums, cnts, w2t, interpret=False):
    return pl.pallas_call(
        _tc_combine_body,
        grid=(N_NODES // M_BLK,),
        in_specs=[
            pl.BlockSpec((M_BLK, D_FEAT), lambda i: (i, 0)),
            pl.BlockSpec((1, M_BLK, DQ), lambda i: (0, i, 0)),
            pl.BlockSpec((1, M_BLK, DQ), lambda i: (1, i, 0)),
            pl.BlockSpec((1, M_BLK, DQ), lambda i: (2, i, 0)),
            pl.BlockSpec((1, M_BLK, DQ), lambda i: (3, i, 0)),
            pl.BlockSpec((1, M_BLK, CNT_W), lambda i: (0, i, 0)),
            pl.BlockSpec((1, M_BLK, CNT_W), lambda i: (1, i, 0)),
            pl.BlockSpec((D_FEAT, D_FEAT), lambda i: (0, 0)),
        ],
        out_specs=pl.BlockSpec((M_BLK, D_FEAT), lambda i: (i, 0)),
        out_shape=jax.ShapeDtypeStruct((N_NODES, D_FEAT), jnp.float32),
        interpret=interpret,
    )(self_part, sums, sums, sums, sums, cnts, cnts, w2t)


def kernel(h, edge_index, W, b, interpret=False):
    src = edge_index[0].astype(jnp.int32)
    dst = edge_index[1].astype(jnp.int32)
    e = src.shape[0]
    pad = E_PAD - e
    src_p = jnp.concatenate([src, jnp.zeros((pad,), jnp.int32)])
    dst_p = jnp.concatenate([dst, jnp.full((pad,), N_NODES, jnp.int32)])
    src3 = src_p.reshape(N_SUBCORES, K, CHUNK)
    dst3 = dst_p.reshape(N_SUBCORES, K, CHUNK)

    sums, cnts = _sc_segment_sum(h, src3, dst3, interpret=interpret)

    wt = W.T
    w1t = wt[:D_FEAT]
    w2t = wt[D_FEAT:]
    b2 = b.reshape(1, D_FEAT)
    self_part = _tc_self(h, w1t, b2, interpret=interpret)
    return _tc_combine(self_part, sums, cnts, w2t, interpret=interpret)


# speedup vs baseline: 1.6289x; 1.0245x over previous
import jax
import jax.numpy as jnp
from jax import lax
from jax.experimental import pallas as pl
from jax.experimental.pallas import tpu as pltpu
from jax.experimental.pallas import tpu_sc as plsc

N_NODES = 10000
D_FEAT = 256
DQ = 64
N_SUBCORES = 16
N_CORES = 2
CHUNK = 125
K = 80
G = 8
NG = K // G
ROWS_PER_TILE = 640
N_PAD = N_SUBCORES * ROWS_PER_TILE
CNT_W = 8
K_SPLIT = 40
RCH = 128
NCH = ROWS_PER_TILE // RCH


def _sc_segment_sum_body(h, src3, dst3, onesz8, z64,
                         sums_out, cnts_out,
                         isrc0, isrc1, idst0, idst1, rows0, rows1, zrows,
                         ones_v, zbuf8,
                         sem_g0, sem_g1, sem_s0, sem_s1, sem_c,
                         htab, acc, cacc):
    c = lax.axis_index("core")
    s = lax.axis_index("subcore")
    base = s * ROWS_PER_TILE
    isrc = (isrc0, isrc1)
    idst = (idst0, idst1)
    rows = (rows0, rows1)
    sem_g = (sem_g0, sem_g1)
    sem_s = (sem_s0, sem_s1)

    pltpu.sync_copy(onesz8.at[0], ones_v)
    pltpu.sync_copy(onesz8.at[1], zbuf8)

    for t in range(2):
        sl = 2 * c + t

        pltpu.sync_copy(z64, zrows)
        tail = N_NODES % RCH

        @pl.loop(0, NCH)
        def _(i):
            off = base + i * RCH
            pltpu.async_copy(zrows, acc.at[pl.ds(off, RCH)], sem_c)
            if t == 0:
                pltpu.async_copy(zbuf8, cacc.at[pl.ds(off, RCH)], sem_c)

        @pl.loop(0, NCH, step=1)
        def _(i):
            off = base + i * RCH

            @pl.when(off + RCH <= N_NODES)
            def _():
                pltpu.async_copy(h.at[pl.ds(off, RCH), pl.ds(sl * DQ, DQ)],
                                 rows0, sem_g0)
                pltpu.make_async_copy(
                    h.at[pl.ds(0, RCH), pl.ds(0, DQ)], rows0,
                    sem_g0).wait()
                pltpu.sync_copy(rows0, htab.at[pl.ds(off, RCH)])

            @pl.when((off < N_NODES) & (off + RCH > N_NODES))
            def _():
                pltpu.sync_copy(h.at[pl.ds(off, tail), pl.ds(sl * DQ, DQ)],
                                rows0.at[pl.ds(0, tail)])
                pltpu.sync_copy(rows0.at[pl.ds(0, tail)],
                                htab.at[pl.ds(off, tail)])

        @pl.loop(0, NCH)
        def _(i):
            pltpu.make_async_copy(zrows, acc.at[pl.ds(base, RCH)],
                                  sem_c).wait()
            if t == 0:
                pltpu.make_async_copy(zbuf8, cacc.at[pl.ds(base, RCH)],
                                      sem_c).wait()
        plsc.subcore_barrier()

        @pl.loop(0, NG, step=2)
        def _(g2):
            for q in range(2):
                gidx = g2 + q
                counting = jnp.where(c == 0, gidx * G < K_SPLIT,
                                     gidx * G >= K_SPLIT)
                pltpu.sync_copy(src3.at[s, pl.ds(gidx * G, G)], isrc[q])
                pltpu.sync_copy(dst3.at[s, pl.ds(gidx * G, G)], idst[q])

                if t == 0:
                    @pl.when(counting)
                    def _():
                        @pl.loop(0, G)
                        def _(j):
                            pltpu.async_copy(ones_v.at[pl.ds(0, CHUNK)],
                                             cacc.at[idst[q].at[j]],
                                             sem_c, add=True)

                @pl.loop(0, G, step=2)
                def _(j0):
                    for p in range(2):
                        j = j0 + p
                        @pl.when(((g2 + q) * G + j0) > 0)
                        def _():
                            pltpu.make_async_copy(
                                rows[p].at[pl.ds(0, CHUNK)],
                                acc.at[pl.ds(base, CHUNK)],
                                sem_s[p]).wait()
                        pltpu.async_copy(htab.at[isrc[q].at[j]],
                                         rows[p].at[pl.ds(0, CHUNK)],
                                         sem_g[p])
                        pltpu.make_async_copy(htab.at[pl.ds(0, CHUNK)],
                                              rows[p].at[pl.ds(0, CHUNK)],
                                              sem_g[p]).wait()
                        pltpu.async_copy(rows[p].at[pl.ds(0, CHUNK)],
                                         acc.at[idst[q].at[j]],
                                         sem_s[p], add=True)

                if t == 0:
                    @pl.when(counting)
                    def _():
                        @pl.loop(0, G)
                        def _(j):
                            pltpu.make_async_copy(
                                ones_v.at[pl.ds(0, CHUNK)],
                                cacc.at[pl.ds(base, CHUNK)],
                                sem_c).wait()

        for p in range(2):
            pltpu.make_async_copy(rows[p].at[pl.ds(0, CHUNK)],
                                  acc.at[pl.ds(base, CHUNK)],
                                  sem_s[p]).wait()
        plsc.subcore_barrier()

        @pl.loop(0, NCH - 1, step=2)
        def _(i0):
            for p in range(2):
                i = i0 + p

                @pl.when(i0 > 0)
                def _():
                    pltpu.make_async_copy(
                        rows[p], sums_out.at[sl, pl.ds(base, RCH)],
                        sem_s[p]).wait()
                off = base + i * RCH
                pltpu.sync_copy(acc.at[pl.ds(off, RCH)], rows[p])
                pltpu.async_copy(rows[p], sums_out.at[sl, pl.ds(off, RCH)],
                                 sem_s[p])
        offl = base + (NCH - 1) * RCH
        pltpu.make_async_copy(rows0, sums_out.at[sl, pl.ds(base, RCH)],
                              sem_s0).wait()
        pltpu.sync_copy(acc.at[pl.ds(offl, RCH)], rows0)
        pltpu.async_copy(rows0, sums_out.at[sl, pl.ds(offl, RCH)], sem_s0)
        if t == 1:
            @pl.loop(0, NCH)
            def _(i):
                off = base + i * RCH
                pltpu.sync_copy(cacc.at[pl.ds(off, RCH)], zbuf8)
                pltpu.sync_copy(zbuf8, cnts_out.at[c, pl.ds(off, RCH)])
        for p in range(2):
            pltpu.make_async_copy(rows[p],
                                  sums_out.at[sl, pl.ds(base, RCH)],
                                  sem_s[p]).wait()


def _sc_segment_sum(h, src3, dst3, interpret=False):
    onesz8 = jnp.stack([jnp.ones((RCH, CNT_W), jnp.float32),
                        jnp.zeros((RCH, CNT_W), jnp.float32)])
    z64 = jnp.zeros((RCH, DQ), jnp.float32)
    fn = pl.kernel(
        _sc_segment_sum_body,
        out_type=(
            jax.ShapeDtypeStruct((4, N_PAD, DQ), jnp.float32),
            jax.ShapeDtypeStruct((N_CORES, N_PAD, CNT_W), jnp.float32),
        ),
        mesh=plsc.VectorSubcoreMesh(core_axis_name="core",
                                    subcore_axis_name="subcore",
                                    num_cores=N_CORES,
                                    num_subcores=N_SUBCORES),
        scratch_types=[
            pltpu.VMEM((G, CHUNK), jnp.int32),
            pltpu.VMEM((G, CHUNK), jnp.int32),
            pltpu.VMEM((G, CHUNK), jnp.int32),
            pltpu.VMEM((G, CHUNK), jnp.int32),
            pltpu.VMEM((RCH, DQ), jnp.float32),
            pltpu.VMEM((RCH, DQ), jnp.float32),
            pltpu.VMEM((RCH, DQ), jnp.float32),
            pltpu.VMEM((RCH, CNT_W), jnp.float32),
            pltpu.VMEM((RCH, CNT_W), jnp.float32),
            pltpu.SemaphoreType.DMA,
            pltpu.SemaphoreType.DMA,
            pltpu.SemaphoreType.DMA,
            pltpu.SemaphoreType.DMA,
            pltpu.SemaphoreType.DMA,
            pltpu.VMEM_SHARED((N_PAD, DQ), jnp.float32),
            pltpu.VMEM_SHARED((N_PAD, DQ), jnp.float32),
            pltpu.VMEM_SHARED((N_PAD, CNT_W), jnp.float32),
        ],
        compiler_params=pltpu.CompilerParams(use_tc_tiling_on_sc=False),
        interpret=interpret,
    )
    return fn(h, src3, dst3, onesz8, z64)


M_BLK = 400


def _tc_self_body(h_ref, w1_ref, b_ref, o_ref):
    o_ref[...] = jnp.dot(h_ref[...], w1_ref[...],
                         preferred_element_type=jnp.float32) + b_ref[...]


def _tc_self(h, w1t, b2, interpret=False):
    return pl.pallas_call(
        _tc_self_body,
        grid=(N_NODES // M_BLK,),
        in_specs=[
            pl.BlockSpec((M_BLK, D_FEAT), lambda i: (i, 0)),
            pl.BlockSpec((D_FEAT, D_FEAT), lambda i: (0, 0)),
            pl.BlockSpec((1, D_FEAT), lambda i: (0, 0)),
        ],
        out_specs=pl.BlockSpec((M_BLK, D_FEAT), lambda i: (i, 0)),
        out_shape=jax.ShapeDtypeStruct((N_NODES, D_FEAT), jnp.float32),
        interpret=interpret,
    )(h, w1t, b2)


def _tc_combine_body(self_ref, s0_ref, s1_ref, s2_ref, s3_ref,
                     c0_ref, c1_ref, w2_ref, o_ref):
    cnt = c0_ref[0][:, 0:1] + c1_ref[0][:, 0:1]
    recip = 1.0 / jnp.maximum(cnt, 1.0)
    agg = jnp.dot(s0_ref[0], w2_ref[pl.ds(0, DQ)],
                  preferred_element_type=jnp.float32)
    agg += jnp.dot(s1_ref[0], w2_ref[pl.ds(DQ, DQ)],
                   preferred_element_type=jnp.float32)
    agg += jnp.dot(s2_ref[0], w2_ref[pl.ds(2 * DQ, DQ)],
                   preferred_element_type=jnp.float32)
    agg += jnp.dot(s3_ref[0], w2_ref[pl.ds(3 * DQ, DQ)],
                   preferred_element_type=jnp.float32)
    o_ref[...] = self_ref[...] + agg * recip


def _tc_combine(self_part, sums, cnts, w2t, interpret=False):
    return pl.pallas_call(
        _tc_combine_body,
        grid=(N_NODES // M_BLK,),
        in_specs=[
            pl.BlockSpec((M_BLK, D_FEAT), lambda i: (i, 0)),
            pl.BlockSpec((1, M_BLK, DQ), lambda i: (0, i, 0)),
            pl.BlockSpec((1, M_BLK, DQ), lambda i: (1, i, 0)),
            pl.BlockSpec((1, M_BLK, DQ), lambda i: (2, i, 0)),
            pl.BlockSpec((1, M_BLK, DQ), lambda i: (3, i, 0)),
            pl.BlockSpec((1, M_BLK, CNT_W), lambda i: (0, i, 0)),
            pl.BlockSpec((1, M_BLK, CNT_W), lambda i: (1, i, 0)),
            pl.BlockSpec((D_FEAT, D_FEAT), lambda i: (0, 0)),
        ],
        out_specs=pl.BlockSpec((M_BLK, D_FEAT), lambda i: (i, 0)),
        out_shape=jax.ShapeDtypeStruct((N_NODES, D_FEAT), jnp.float32),
        interpret=interpret,
    )(self_part, sums, sums, sums, sums, cnts, cnts, w2t)


def kernel(h, edge_index, W, b, interpret=False):
    src3 = edge_index[0].astype(jnp.int32).reshape(N_SUBCORES, K, CHUNK)
    dst3 = edge_index[1].astype(jnp.int32).reshape(N_SUBCORES, K, CHUNK)

    sums, cnts = _sc_segment_sum(h, src3, dst3, interpret=interpret)

    wt = W.T
    w1t = wt[:D_FEAT]
    w2t = wt[D_FEAT:]
    b2 = b.reshape(1, D_FEAT)
    self_part = _tc_self(h, w1t, b2, interpret=interpret)
    return _tc_combine(self_part, sums, cnts, w2t, interpret=interpret)


# speedup vs baseline: 1.6315x; 1.0016x over previous
import jax
import jax.numpy as jnp
from jax import lax
from jax.experimental import pallas as pl
from jax.experimental.pallas import tpu as pltpu
from jax.experimental.pallas import tpu_sc as plsc

N_NODES = 10000
D_FEAT = 256
DQ = 64
N_SUBCORES = 16
N_CORES = 2
CHUNK = 125
K = 80
G = 8
NG = K // G
ROWS_PER_TILE = 640
N_PAD = N_SUBCORES * ROWS_PER_TILE
CNT_W = 8
K_SPLIT = 40
RCH = 128
NCH = ROWS_PER_TILE // RCH


def _sc_segment_sum_body(h, src3, dst3, onesz8, z64,
                         sums_out, cnts_out,
                         isrc0, isrc1, idst0, idst1, rows0, rows1, zrows,
                         ones_v, zbuf8,
                         sem_g0, sem_g1, sem_s0, sem_s1, sem_c,
                         htab, acc, cacc):
    c = lax.axis_index("core")
    s = lax.axis_index("subcore")
    base = s * ROWS_PER_TILE
    isrc = (isrc0, isrc1)
    idst = (idst0, idst1)
    rows = (rows0, rows1)
    sem_g = (sem_g0, sem_g1)
    sem_s = (sem_s0, sem_s1)

    pltpu.sync_copy(onesz8.at[0], ones_v)
    pltpu.sync_copy(onesz8.at[1], zbuf8)

    for t in range(2):
        sl = 2 * c + t

        pltpu.sync_copy(z64, zrows)
        tail = N_NODES % RCH

        @pl.loop(0, NCH)
        def _(i):
            off = base + i * RCH
            pltpu.async_copy(zrows, acc.at[pl.ds(off, RCH)], sem_c)
            if t == 0:
                pltpu.async_copy(zbuf8, cacc.at[pl.ds(off, RCH)], sem_c)

        @pl.loop(0, NCH, step=1)
        def _(i):
            off = base + i * RCH

            @pl.when(off + RCH <= N_NODES)
            def _():
                pltpu.async_copy(h.at[pl.ds(off, RCH), pl.ds(sl * DQ, DQ)],
                                 rows0, sem_g0)
                pltpu.make_async_copy(
                    h.at[pl.ds(0, RCH), pl.ds(0, DQ)], rows0,
                    sem_g0).wait()
                pltpu.sync_copy(rows0, htab.at[pl.ds(off, RCH)])

            @pl.when((off < N_NODES) & (off + RCH > N_NODES))
            def _():
                pltpu.sync_copy(h.at[pl.ds(off, tail), pl.ds(sl * DQ, DQ)],
                                rows0.at[pl.ds(0, tail)])
                pltpu.sync_copy(rows0.at[pl.ds(0, tail)],
                                htab.at[pl.ds(off, tail)])

        @pl.loop(0, NCH)
        def _(i):
            pltpu.make_async_copy(zrows, acc.at[pl.ds(base, RCH)],
                                  sem_c).wait()
            if t == 0:
                pltpu.make_async_copy(zbuf8, cacc.at[pl.ds(base, RCH)],
                                      sem_c).wait()
        plsc.subcore_barrier()

        @pl.loop(0, NG, step=2)
        def _(g2):
            for q in range(2):
                gidx = g2 + q
                counting = jnp.where(c == 0, gidx * G < K_SPLIT,
                                     gidx * G >= K_SPLIT)
                pltpu.sync_copy(src3.at[s, pl.ds(gidx * G, G)], isrc[q])
                pltpu.sync_copy(dst3.at[s, pl.ds(gidx * G, G)], idst[q])

                if t == 0:
                    @pl.when(counting)
                    def _():
                        @pl.loop(0, G)
                        def _(j):
                            pltpu.async_copy(ones_v.at[pl.ds(0, CHUNK)],
                                             cacc.at[idst[q].at[j]],
                                             sem_c, add=True)

                @pl.loop(0, G, step=2)
                def _(j0):
                    for p in range(2):
                        j = j0 + p
                        @pl.when(((g2 + q) * G + j0) > 0)
                        def _():
                            pltpu.make_async_copy(
                                rows[p].at[pl.ds(0, CHUNK)],
                                acc.at[pl.ds(base, CHUNK)],
                                sem_s[p]).wait()
                        pltpu.async_copy(htab.at[isrc[q].at[j]],
                                         rows[p].at[pl.ds(0, CHUNK)],
                                         sem_g[p])
                        pltpu.make_async_copy(htab.at[pl.ds(0, CHUNK)],
                                              rows[p].at[pl.ds(0, CHUNK)],
                                              sem_g[p]).wait()
                        pltpu.async_copy(rows[p].at[pl.ds(0, CHUNK)],
                                         acc.at[idst[q].at[j]],
                                         sem_s[p], add=True)

                if t == 0:
                    @pl.when(counting)
                    def _():
                        @pl.loop(0, G)
                        def _(j):
                            pltpu.make_async_copy(
                                ones_v.at[pl.ds(0, CHUNK)],
                                cacc.at[pl.ds(base, CHUNK)],
                                sem_c).wait()

        for p in range(2):
            pltpu.make_async_copy(rows[p].at[pl.ds(0, CHUNK)],
                                  acc.at[pl.ds(base, CHUNK)],
                                  sem_s[p]).wait()
        plsc.subcore_barrier()

        @pl.loop(0, NCH - 1, step=2)
        def _(i0):
            for p in range(2):
                i = i0 + p

                @pl.when(i0 > 0)
                def _():
                    pltpu.make_async_copy(
                        rows[p], sums_out.at[sl, pl.ds(base, RCH)],
                        sem_s[p]).wait()
                off = base + i * RCH
                pltpu.sync_copy(acc.at[pl.ds(off, RCH)], rows[p])
                pltpu.async_copy(rows[p], sums_out.at[sl, pl.ds(off, RCH)],
                                 sem_s[p])
        offl = base + (NCH - 1) * RCH
        pltpu.make_async_copy(rows0, sums_out.at[sl, pl.ds(base, RCH)],
                              sem_s0).wait()
        pltpu.sync_copy(acc.at[pl.ds(offl, RCH)], rows0)
        pltpu.async_copy(rows0, sums_out.at[sl, pl.ds(offl, RCH)], sem_s0)
        if t == 1:
            @pl.loop(0, NCH)
            def _(i):
                off = base + i * RCH
                pltpu.sync_copy(cacc.at[pl.ds(off, RCH)], zbuf8)
                pltpu.sync_copy(zbuf8, cnts_out.at[c, pl.ds(off, RCH)])
        for p in range(2):
            pltpu.make_async_copy(rows[p],
                                  sums_out.at[sl, pl.ds(base, RCH)],
                                  sem_s[p]).wait()


def _sc_segment_sum(h, src3, dst3, interpret=False):
    onesz8 = jnp.stack([jnp.ones((RCH, CNT_W), jnp.float32),
                        jnp.zeros((RCH, CNT_W), jnp.float32)])
    z64 = jnp.zeros((RCH, DQ), jnp.float32)
    fn = pl.kernel(
        _sc_segment_sum_body,
        out_type=(
            jax.ShapeDtypeStruct((4, N_PAD, DQ), jnp.float32),
            jax.ShapeDtypeStruct((N_CORES, N_PAD, CNT_W), jnp.float32),
        ),
        mesh=plsc.VectorSubcoreMesh(core_axis_name="core",
                                    subcore_axis_name="subcore",
                                    num_cores=N_CORES,
                                    num_subcores=N_SUBCORES),
        scratch_types=[
            pltpu.VMEM((G, CHUNK), jnp.int32),
            pltpu.VMEM((G, CHUNK), jnp.int32),
            pltpu.VMEM((G, CHUNK), jnp.int32),
            pltpu.VMEM((G, CHUNK), jnp.int32),
            pltpu.VMEM((RCH, DQ), jnp.float32),
            pltpu.VMEM((RCH, DQ), jnp.float32),
            pltpu.VMEM((RCH, DQ), jnp.float32),
            pltpu.VMEM((RCH, CNT_W), jnp.float32),
            pltpu.VMEM((RCH, CNT_W), jnp.float32),
            pltpu.SemaphoreType.DMA,
            pltpu.SemaphoreType.DMA,
            pltpu.SemaphoreType.DMA,
            pltpu.SemaphoreType.DMA,
            pltpu.SemaphoreType.DMA,
            pltpu.VMEM_SHARED((N_PAD, DQ), jnp.float32),
            pltpu.VMEM_SHARED((N_PAD, DQ), jnp.float32),
            pltpu.VMEM_SHARED((N_PAD, CNT_W), jnp.float32),
        ],
        compiler_params=pltpu.CompilerParams(use_tc_tiling_on_sc=False),
        interpret=interpret,
    )
    return fn(h, src3, dst3, onesz8, z64)


M_BLK = 400


def _tc_self_body(h_ref, w1_ref, b_ref, o_ref):
    o_ref[...] = jnp.dot(h_ref[...], w1_ref[...],
                         preferred_element_type=jnp.float32) + b_ref[...]


def _tc_self(h, w1t, b2, interpret=False):
    return pl.pallas_call(
        _tc_self_body,
        grid=(N_NODES // M_BLK,),
        in_specs=[
            pl.BlockSpec((M_BLK, D_FEAT), lambda i: (i, 0)),
            pl.BlockSpec((D_FEAT, D_FEAT), lambda i: (0, 0)),
            pl.BlockSpec((1, D_FEAT), lambda i: (0, 0)),
        ],
        out_specs=pl.BlockSpec((M_BLK, D_FEAT), lambda i: (i, 0)),
        out_shape=jax.ShapeDtypeStruct((N_NODES, D_FEAT), jnp.float32),
        interpret=interpret,
    )(h, w1t, b2)


def _tc_combine_body(self_ref, s0_ref, s1_ref, s2_ref, s3_ref,
                     c0_ref, c1_ref, w2_ref, o_ref):
    cnt = c0_ref[0][:, 0:1] + c1_ref[0][:, 0:1]
    recip = 1.0 / jnp.maximum(cnt, 1.0)
    agg = jnp.dot(s0_ref[0], w2_ref[pl.ds(0, DQ)],
                  preferred_element_type=jnp.float32)
    agg += jnp.dot(s1_ref[0], w2_ref[pl.ds(DQ, DQ)],
                   preferred_element_type=jnp.float32)
    agg += jnp.dot(s2_ref[0], w2_ref[pl.ds(2 * DQ, DQ)],
                   preferred_element_type=jnp.float32)
    agg += jnp.dot(s3_ref[0], w2_ref[pl.ds(3 * DQ, DQ)],
                   preferred_element_type=jnp.float32)
    o_ref[...] = self_ref[...] + agg * recip


def _tc_combine(self_part, sums, cnts, w2t, interpret=False):
    return pl.pallas_call(
        _tc_combine_body,
        grid=(N_NODES // M_BLK,),
        in_specs=[
            pl.BlockSpec((M_BLK, D_FEAT), lambda i: (i, 0)),
            pl.BlockSpec((1, M_BLK, DQ), lambda i: (0, i, 0)),
            pl.BlockSpec((1, M_BLK, DQ), lambda i: (1, i, 0)),
            pl.BlockSpec((1, M_BLK, DQ), lambda i: (2, i, 0)),
            pl.BlockSpec((1, M_BLK, DQ), lambda i: (3, i, 0)),
            pl.BlockSpec((1, M_BLK, CNT_W), lambda i: (0, i, 0)),
            pl.BlockSpec((1, M_BLK, CNT_W), lambda i: (1, i, 0)),
            pl.BlockSpec((D_FEAT, D_FEAT), lambda i: (0, 0)),
        ],
        out_specs=pl.BlockSpec((M_BLK, D_FEAT), lambda i: (i, 0)),
        out_shape=jax.ShapeDtypeStruct((N_NODES, D_FEAT), jnp.float32),
        interpret=interpret,
    )(self_part, sums, sums, sums, sums, cnts, cnts, w2t)


def kernel(h, edge_index, W, b):
    src3 = edge_index[0].astype(jnp.int32).reshape(N_SUBCORES, K, CHUNK)
    dst3 = edge_index[1].astype(jnp.int32).reshape(N_SUBCORES, K, CHUNK)

    sums, cnts = _sc_segment_sum(h, src3, dst3)

    wt = W.T
    w1t = wt[:D_FEAT]
    w2t = wt[D_FEAT:]
    b2 = b.reshape(1, D_FEAT)
    self_part = _tc_self(h, w1t, b2)
    return _tc_combine(self_part, sums, cnts, w2t)
